# initial kernel scaffold (unmeasured)
import jax
import jax.numpy as jnp
from jax import lax
from jax.experimental import pallas as pl
from jax.experimental.pallas import tpu as pltpu


def kernel(
    x,
):
    def body(*refs):
        pass

    out_shape = jax.ShapeDtypeStruct(..., jnp.float32)
    return pl.pallas_call(body, out_shape=out_shape)(...)


# baseline (device time: 7376 ns/iter reference)
import functools

import jax
import jax.numpy as jnp
from jax import lax
from jax.experimental import pallas as pl
from jax.experimental.pallas import tpu as pltpu

MESH_X = 2
MESH_Y = 2


def kernel(x):
    m, n = x.shape

    def body(
        x_ref, out_ref, row_send, row_recv, col_send, col_recv,
        send_sems, recv_sems,
    ):
        my_x = lax.axis_index("x")
        my_y = lax.axis_index("y")
        nbr_x = MESH_X - 1 - my_x
        nbr_y = MESH_Y - 1 - my_y

        xv = x_ref[:, :]

        row_send[:, :] = jnp.where(my_x == 0, xv[m - 1 : m, :], xv[0:1, :])
        colv = jnp.where(my_y == 0, xv[:, n - 1 : n], xv[:, 0:1])
        col_send[:, :] = jnp.transpose(colv, (1, 0))

        barrier_sem = pltpu.get_barrier_semaphore()
        pl.semaphore_signal(
            barrier_sem, inc=1,
            device_id=(nbr_x, my_y), device_id_type=pl.DeviceIdType.MESH,
        )
        pl.semaphore_signal(
            barrier_sem, inc=1,
            device_id=(my_x, nbr_y), device_id_type=pl.DeviceIdType.MESH,
        )
        pl.semaphore_wait(barrier_sem, 2)

        rdma_row = pltpu.make_async_remote_copy(
            src_ref=row_send,
            dst_ref=row_recv,
            send_sem=send_sems.at[0],
            recv_sem=recv_sems.at[0],
            device_id=(nbr_x, my_y),
            device_id_type=pl.DeviceIdType.MESH,
        )
        rdma_col = pltpu.make_async_remote_copy(
            src_ref=col_send,
            dst_ref=col_recv,
            send_sem=send_sems.at[1],
            recv_sem=recv_sems.at[1],
            device_id=(my_x, nbr_y),
            device_id_type=pl.DeviceIdType.MESH,
        )
        rdma_row.start()
        rdma_col.start()
        rdma_row.wait()
        rdma_col.wait()

        rbuf = row_recv[:, :]
        cbuf = jnp.transpose(col_recv[:, :], (1, 0))

        up = jnp.concatenate([rbuf, xv[:-1, :]], axis=0)
        down = jnp.concatenate([xv[1:, :], rbuf], axis=0)
        left = jnp.concatenate([cbuf, xv[:, :-1]], axis=1)
        right = jnp.concatenate([xv[:, 1:], cbuf], axis=1)

        stencil = 0.5 * xv + 0.125 * (up + down + left + right)

        gr = lax.broadcasted_iota(jnp.int32, (m, n), 0) + my_x * m
        gc = lax.broadcasted_iota(jnp.int32, (m, n), 1) + my_y * n
        boundary = (
            (gr == 0) | (gr == MESH_X * m - 1) | (gc == 0) | (gc == MESH_Y * n - 1)
        )
        out_ref[:, :] = jnp.where(boundary, xv, stencil)

        @functools.partial(
            pl.run_scoped, second_barrier=pltpu.SemaphoreType.REGULAR
        )
        def _(second_barrier):
            pl.semaphore_signal(
                second_barrier, inc=1,
                device_id=(nbr_x, my_y), device_id_type=pl.DeviceIdType.MESH,
            )
            pl.semaphore_signal(
                second_barrier, inc=1,
                device_id=(my_x, nbr_y), device_id_type=pl.DeviceIdType.MESH,
            )
            pl.semaphore_wait(second_barrier, 2)

    return pl.pallas_call(
        body,
        out_shape=jax.ShapeDtypeStruct((m, n), x.dtype),
        in_specs=[pl.BlockSpec(memory_space=pltpu.VMEM)],
        out_specs=pl.BlockSpec(memory_space=pltpu.VMEM),
        scratch_shapes=[
            pltpu.VMEM((1, n), x.dtype),
            pltpu.VMEM((1, n), x.dtype),
            pltpu.VMEM((1, m), x.dtype),
            pltpu.VMEM((1, m), x.dtype),
            pltpu.SemaphoreType.DMA((2,)),
            pltpu.SemaphoreType.DMA((2,)),
        ],
        compiler_params=pltpu.CompilerParams(collective_id=0),
    )(x)
